# Initial kernel scaffold; baseline (speedup 1.0000x reference)
import jax
import jax.numpy as jnp
from jax.experimental import pallas as pl


def kernel(input_ids, pinyin_ids, word_emb, pinyin_emb, conv_w, conv_b, glyph_emb, glyph_map_w, glyph_map_b, map_fc_w, map_fc_b, pos_emb, tt_emb, ln_g, ln_b):
    raise NotImplementedError("write your pallas kernel here")


# speedup vs baseline: 1.0717x; 1.0717x over previous
import functools

import jax
import jax.numpy as jnp
from jax import lax
from jax.experimental import pallas as pl
from jax.experimental.pallas import tpu as pltpu
from jax.experimental.pallas import tpu_sc as plsc

B, S = 32, 512
N = B * S
H = 768
GD = 1728
PV, PE, PL = 32, 128, 8
EPS = 1e-12

NC, NS = 2, 16
NW = NC * NS
TOK_PER_W = N // NW
CHUNK = 32
NCH = TOK_PER_W // CHUNK


def _sc_gather(word_emb, glyph_emb, ids):
  mesh = plsc.VectorSubcoreMesh(core_axis_name="c", subcore_axis_name="s")

  @functools.partial(
      pl.kernel, mesh=mesh,
      compiler_params=pltpu.CompilerParams(use_tc_tiling_on_sc=False),
      out_type=[jax.ShapeDtypeStruct((N, H), jnp.float32),
                jax.ShapeDtypeStruct((N, GD), jnp.float32)],
      scratch_types=[
          pltpu.VMEM((NCH, CHUNK), jnp.int32),
          pltpu.VMEM((CHUNK, H), jnp.float32),
          pltpu.VMEM((CHUNK, GD), jnp.float32),
          pltpu.SemaphoreType.DMA,
      ],
  )
  def k(word_hbm, glyph_hbm, ids_hbm, outw_hbm, outg_hbm,
        idx_v, roww_v, rowg_v, sem):
    wid = lax.axis_index("s") * NC + lax.axis_index("c")
    base = wid * TOK_PER_W
    pltpu.sync_copy(ids_hbm.at[pl.ds(wid * NCH, NCH)], idx_v)
    for c in range(NCH):
      pltpu.async_copy(word_hbm.at[idx_v.at[c]], roww_v, sem).wait()
      pltpu.sync_copy(roww_v, outw_hbm.at[pl.ds(base + c * CHUNK, CHUNK)])
    for c in range(NCH):
      pltpu.async_copy(glyph_hbm.at[idx_v.at[c]], rowg_v, sem).wait()
      pltpu.sync_copy(rowg_v, outg_hbm.at[pl.ds(base + c * CHUNK, CHUNK)])

  return k(word_emb, glyph_emb, ids)


def _prep_body(gmw_ref, wc_ref, gmb_ref, fcb_ref, tt0_ref, pemb_ref,
               cw0_ref, cw1_ref, gw_ref, t01_ref, const_ref):
  gw = jnp.dot(gmw_ref[...].astype(jnp.bfloat16), wc_ref[...].astype(jnp.bfloat16),
               preferred_element_type=jnp.float32)
  gw_ref[...] = gw.astype(jnp.bfloat16)
  t0 = jnp.dot(pemb_ref[...], cw0_ref[...], preferred_element_type=jnp.float32)
  t1 = jnp.dot(pemb_ref[...], cw1_ref[...], preferred_element_type=jnp.float32)
  t01_ref[...] = jnp.concatenate([t0, t1], axis=0).astype(jnp.bfloat16)
  const_ref[...] = (jnp.dot(gmb_ref[...], wc_ref[...],
                            preferred_element_type=jnp.float32)
                    + fcb_ref[...] + tt0_ref[...])


def _prep(glyph_map_w, wc, gmb, fcb, tt0, pinyin_emb, cw0, cw1):
  return pl.pallas_call(
      _prep_body,
      out_shape=[jax.ShapeDtypeStruct((GD, H), jnp.bfloat16),
                 jax.ShapeDtypeStruct((2 * PV, H), jnp.bfloat16),
                 jax.ShapeDtypeStruct((1, H), jnp.float32)],
  )(glyph_map_w, wc, gmb, fcb, tt0, pinyin_emb, cw0, cw1)


def _main_body(wg_ref, gg_ref, pid_ref, wa_ref, wb_ref, gw_ref, t01_ref,
               convb_ref, const_ref, pos_ref, lng_ref, lnb_ref, out_ref):
  pid = pid_ref[...]
  tb = pid.shape[0]
  iot = lax.broadcasted_iota(jnp.int32, (tb, PV), 1)
  oh = [(pid[:, j:j + 1] == iot).astype(jnp.bfloat16) for j in range(PL)]
  t01 = t01_ref[...]
  m = None
  for j in range(PL - 1):
    x = jnp.dot(jnp.concatenate([oh[j], oh[j + 1]], axis=1), t01,
                preferred_element_type=jnp.float32)
    m = x if m is None else jnp.maximum(m, x)
  py = m + convb_ref[...]
  acc = jnp.dot(wg_ref[...].astype(jnp.bfloat16), wa_ref[...],
                preferred_element_type=jnp.float32)
  acc = acc + jnp.dot(py.astype(jnp.bfloat16), wb_ref[...],
                      preferred_element_type=jnp.float32)
  acc = acc + jnp.dot(gg_ref[...].astype(jnp.bfloat16), gw_ref[...],
                      preferred_element_type=jnp.float32)
  acc = acc + const_ref[...] + pos_ref[...]
  mu = jnp.mean(acc, axis=1, keepdims=True)
  d = acc - mu
  var = jnp.mean(d * d, axis=1, keepdims=True)
  out_ref[...] = d * lax.rsqrt(var + EPS) * lng_ref[...] + lnb_ref[...]


def _main(wg, gg, pid2d, wa_bf, wb_bf, gw_bf, t01_bf, convb, constv,
          pos_emb, lng, lnb):
  tb = S
  grid = (N // tb,)
  return pl.pallas_call(
      _main_body,
      grid=grid,
      in_specs=[
          pl.BlockSpec((tb, H), lambda i: (i, 0)),
          pl.BlockSpec((tb, GD), lambda i: (i, 0)),
          pl.BlockSpec((tb, PL), lambda i: (i, 0)),
          pl.BlockSpec((H, H), lambda i: (0, 0)),
          pl.BlockSpec((H, H), lambda i: (0, 0)),
          pl.BlockSpec((GD, H), lambda i: (0, 0)),
          pl.BlockSpec((2 * PV, H), lambda i: (0, 0)),
          pl.BlockSpec((1, H), lambda i: (0, 0)),
          pl.BlockSpec((1, H), lambda i: (0, 0)),
          pl.BlockSpec((S, H), lambda i: (0, 0)),
          pl.BlockSpec((1, H), lambda i: (0, 0)),
          pl.BlockSpec((1, H), lambda i: (0, 0)),
      ],
      out_specs=pl.BlockSpec((tb, H), lambda i: (i, 0)),
      out_shape=jax.ShapeDtypeStruct((N, H), jnp.float32),
  )(wg, gg, pid2d, wa_bf, wb_bf, gw_bf, t01_bf, convb, constv,
    pos_emb, lng, lnb)


def kernel(input_ids, pinyin_ids, word_emb, pinyin_emb, conv_w, conv_b,
           glyph_emb, glyph_map_w, glyph_map_b, map_fc_w, map_fc_b,
           pos_emb, tt_emb, ln_g, ln_b):
  wg, gg = _sc_gather(word_emb, glyph_emb, input_ids.reshape(N // CHUNK, CHUNK))

  gw_bf, t01_bf, constv = _prep(
      glyph_map_w, map_fc_w[2 * H:], glyph_map_b.reshape(1, H),
      map_fc_b.reshape(1, H), tt_emb[0:1], pinyin_emb, conv_w[0], conv_w[1])

  wa_bf = map_fc_w[:H].astype(jnp.bfloat16)
  wb_bf = map_fc_w[H:2 * H].astype(jnp.bfloat16)
  out = _main(wg, gg, pinyin_ids.reshape(N, PL), wa_bf, wb_bf, gw_bf,
              t01_bf, conv_b.reshape(1, H), constv, pos_emb,
              ln_g.reshape(1, H), ln_b.reshape(1, H))
  return out.reshape(B, S, H)


# speedup vs baseline: 3.1269x; 2.9177x over previous
import functools

import jax
import jax.numpy as jnp
from jax import lax
from jax.experimental import pallas as pl
from jax.experimental.pallas import tpu as pltpu
from jax.experimental.pallas import tpu_sc as plsc

B, S = 32, 512
N = B * S
H = 768
V = 23236
GD = 1728
PV, PE, PL = 32, 128, 8
EPS = 1e-12

NC, NS = 2, 16
NW = NC * NS
TOK_PER_W = N // NW
CHUNK = 64
NCH = TOK_PER_W // CHUNK

VB = 512
NVB = (V + VB - 1) // VB


def _sc_gather(table, ids):
  mesh = plsc.VectorSubcoreMesh(core_axis_name="c", subcore_axis_name="s")

  @functools.partial(
      pl.kernel, mesh=mesh,
      out_type=jax.ShapeDtypeStruct((N, H), jnp.float32),
      scratch_types=[
          pltpu.VMEM((NCH, CHUNK), jnp.int32),
          pltpu.VMEM((CHUNK, H), jnp.float32),
          pltpu.SemaphoreType.DMA,
      ],
  )
  def k(table_hbm, ids_hbm, out_hbm, idx_v, rows_v, sem):
    wid = lax.axis_index("s") * NC + lax.axis_index("c")
    base = wid * TOK_PER_W
    pltpu.sync_copy(ids_hbm.at[pl.ds(wid * NCH, NCH)], idx_v)
    for c in range(NCH):
      pltpu.async_copy(table_hbm.at[idx_v.at[c]], rows_v, sem).wait()
      pltpu.sync_copy(rows_v, out_hbm.at[pl.ds(base + c * CHUNK, CHUNK)])

  return k(table, ids)


def _prep_body(gmw_ref, wc_ref, gmb_ref, fcb_ref, tt0_ref, pemb_ref,
               cw0_ref, cw1_ref, gw_ref, t01_ref, const_ref):
  gw = jnp.dot(gmw_ref[...].astype(jnp.bfloat16), wc_ref[...].astype(jnp.bfloat16),
               preferred_element_type=jnp.float32)
  gw_ref[...] = gw.astype(jnp.bfloat16)
  t0 = jnp.dot(pemb_ref[...], cw0_ref[...], preferred_element_type=jnp.float32)
  t1 = jnp.dot(pemb_ref[...], cw1_ref[...], preferred_element_type=jnp.float32)
  t01_ref[...] = jnp.concatenate([t0, t1], axis=0).astype(jnp.bfloat16)
  const_ref[...] = (jnp.dot(gmb_ref[...], wc_ref[...],
                            preferred_element_type=jnp.float32)
                    + fcb_ref[...] + tt0_ref[...])


def _prep(glyph_map_w, wc, gmb, fcb, tt0, pinyin_emb, cw0, cw1):
  return pl.pallas_call(
      _prep_body,
      out_shape=[jax.ShapeDtypeStruct((GD, H), jnp.bfloat16),
                 jax.ShapeDtypeStruct((2 * PV, H), jnp.bfloat16),
                 jax.ShapeDtypeStruct((1, H), jnp.float32)],
  )(glyph_map_w, wc, gmb, fcb, tt0, pinyin_emb, cw0, cw1)


def _proj_body(we_ref, ge_ref, wa_ref, gw_ref, const_ref, out_ref):
  acc = jnp.dot(we_ref[...].astype(jnp.bfloat16), wa_ref[...],
                preferred_element_type=jnp.float32)
  acc = acc + jnp.dot(ge_ref[...].astype(jnp.bfloat16), gw_ref[...],
                      preferred_element_type=jnp.float32)
  out_ref[...] = acc + const_ref[...]


def _proj(word_emb, glyph_emb, wa_bf, gw_bf, constv):
  return pl.pallas_call(
      _proj_body,
      grid=(NVB,),
      in_specs=[
          pl.BlockSpec((VB, H), lambda i: (i, 0)),
          pl.BlockSpec((VB, GD), lambda i: (i, 0)),
          pl.BlockSpec((H, H), lambda i: (0, 0)),
          pl.BlockSpec((GD, H), lambda i: (0, 0)),
          pl.BlockSpec((1, H), lambda i: (0, 0)),
      ],
      out_specs=pl.BlockSpec((VB, H), lambda i: (i, 0)),
      out_shape=jax.ShapeDtypeStruct((V, H), jnp.float32),
  )(word_emb, glyph_emb, wa_bf, gw_bf, constv)


def _main_body(pg_ref, pid_ref, wb_ref, t01_ref, convb_ref, pos_ref,
               lng_ref, lnb_ref, out_ref):
  pid = pid_ref[...]
  iot = lax.broadcasted_iota(jnp.int32, (S, PV), 1)
  oh = [(pid[:, j:j + 1] == iot).astype(jnp.bfloat16) for j in range(PL)]
  t01 = t01_ref[...]
  m = None
  for j in range(PL - 1):
    x = jnp.dot(jnp.concatenate([oh[j], oh[j + 1]], axis=1), t01,
                preferred_element_type=jnp.float32)
    m = x if m is None else jnp.maximum(m, x)
  py = m + convb_ref[...]
  acc = pg_ref[...] + jnp.dot(py.astype(jnp.bfloat16), wb_ref[...],
                              preferred_element_type=jnp.float32)
  acc = acc + pos_ref[...]
  mu = jnp.mean(acc, axis=1, keepdims=True)
  d = acc - mu
  var = jnp.mean(d * d, axis=1, keepdims=True)
  out_ref[...] = d * lax.rsqrt(var + EPS) * lng_ref[...] + lnb_ref[...]


def _main(pg, pid2d, wb_bf, t01_bf, convb, pos_emb, lng, lnb):
  return pl.pallas_call(
      _main_body,
      grid=(N // S,),
      in_specs=[
          pl.BlockSpec((S, H), lambda i: (i, 0)),
          pl.BlockSpec((S, PL), lambda i: (i, 0)),
          pl.BlockSpec((H, H), lambda i: (0, 0)),
          pl.BlockSpec((2 * PV, H), lambda i: (0, 0)),
          pl.BlockSpec((1, H), lambda i: (0, 0)),
          pl.BlockSpec((S, H), lambda i: (0, 0)),
          pl.BlockSpec((1, H), lambda i: (0, 0)),
          pl.BlockSpec((1, H), lambda i: (0, 0)),
      ],
      out_specs=pl.BlockSpec((S, H), lambda i: (i, 0)),
      out_shape=jax.ShapeDtypeStruct((N, H), jnp.float32),
  )(pg, pid2d, wb_bf, t01_bf, convb, pos_emb, lng, lnb)


def kernel(input_ids, pinyin_ids, word_emb, pinyin_emb, conv_w, conv_b,
           glyph_emb, glyph_map_w, glyph_map_b, map_fc_w, map_fc_b,
           pos_emb, tt_emb, ln_g, ln_b):
  gw_bf, t01_bf, constv = _prep(
      glyph_map_w, map_fc_w[2 * H:], glyph_map_b.reshape(1, H),
      map_fc_b.reshape(1, H), tt_emb[0:1], pinyin_emb, conv_w[0], conv_w[1])

  wa_bf = map_fc_w[:H].astype(jnp.bfloat16)
  wb_bf = map_fc_w[H:2 * H].astype(jnp.bfloat16)

  ptab = _proj(word_emb, glyph_emb, wa_bf, gw_bf, constv)
  pg = _sc_gather(ptab, input_ids.reshape(N // CHUNK, CHUNK))

  out = _main(pg, pinyin_ids.reshape(N, PL), wb_bf, t01_bf,
              conv_b.reshape(1, H), pos_emb,
              ln_g.reshape(1, H), ln_b.reshape(1, H))
  return out.reshape(B, S, H)


# speedup vs baseline: 3.2686x; 1.0453x over previous
import functools

import jax
import jax.numpy as jnp
from jax import lax
from jax.experimental import pallas as pl
from jax.experimental.pallas import tpu as pltpu
from jax.experimental.pallas import tpu_sc as plsc

B, S = 32, 512
N = B * S
H = 768
V = 23236
GD = 1728
PV, PE, PL = 32, 128, 8
EPS = 1e-12

NC, NS = 2, 16
NW = NC * NS
TOK_PER_W = N // NW
CHUNK = 64
NCH = TOK_PER_W // CHUNK

VB = 512
NVB = (V + VB - 1) // VB


def _sc_gather(table, ids):
  mesh = plsc.VectorSubcoreMesh(core_axis_name="c", subcore_axis_name="s")

  @functools.partial(
      pl.kernel, mesh=mesh,
      out_type=jax.ShapeDtypeStruct((N, H), jnp.float32),
      scratch_types=[
          pltpu.VMEM((NCH, CHUNK), jnp.int32),
          pltpu.VMEM((CHUNK, H), jnp.float32),
          pltpu.SemaphoreType.DMA,
      ],
  )
  def k(table_hbm, ids_hbm, out_hbm, idx_v, rows_v, sem):
    wid = lax.axis_index("s") * NC + lax.axis_index("c")
    base = wid * TOK_PER_W
    pltpu.sync_copy(ids_hbm.at[pl.ds(wid * NCH, NCH)], idx_v)
    for c in range(NCH):
      pltpu.async_copy(table_hbm.at[idx_v.at[c]], rows_v, sem).wait()
      pltpu.sync_copy(rows_v, out_hbm.at[pl.ds(base + c * CHUNK, CHUNK)])

  return k(table, ids)


def _prep_body(gmw_ref, wc_ref, gmb_ref, fcb_ref, tt0_ref, pemb_ref,
               cw0_ref, cw1_ref, gw_ref, t01_ref, const_ref):
  gw = jnp.dot(gmw_ref[...].astype(jnp.bfloat16), wc_ref[...].astype(jnp.bfloat16),
               preferred_element_type=jnp.float32)
  gw_ref[...] = gw.astype(jnp.bfloat16)
  t0 = jnp.dot(pemb_ref[...], cw0_ref[...], preferred_element_type=jnp.float32)
  t1 = jnp.dot(pemb_ref[...], cw1_ref[...], preferred_element_type=jnp.float32)
  t01_ref[...] = jnp.concatenate([t0, t1], axis=0).astype(jnp.bfloat16)
  const_ref[...] = (jnp.dot(gmb_ref[...], wc_ref[...],
                            preferred_element_type=jnp.float32)
                    + fcb_ref[...] + tt0_ref[...])


def _prep(glyph_map_w, wc, gmb, fcb, tt0, pinyin_emb, cw0, cw1):
  return pl.pallas_call(
      _prep_body,
      out_shape=[jax.ShapeDtypeStruct((GD, H), jnp.bfloat16),
                 jax.ShapeDtypeStruct((2 * PV, H), jnp.bfloat16),
                 jax.ShapeDtypeStruct((1, H), jnp.float32)],
  )(glyph_map_w, wc, gmb, fcb, tt0, pinyin_emb, cw0, cw1)


def _proj_body(we_ref, ge_ref, wa_ref, gw_ref, const_ref, out_ref):
  acc = jnp.dot(we_ref[...].astype(jnp.bfloat16), wa_ref[...],
                preferred_element_type=jnp.float32)
  acc = acc + jnp.dot(ge_ref[...].astype(jnp.bfloat16), gw_ref[...],
                      preferred_element_type=jnp.float32)
  out_ref[...] = acc + const_ref[...]


def _proj(word_emb, glyph_emb, wa_bf, gw_bf, constv):
  return pl.pallas_call(
      _proj_body,
      grid=(NVB,),
      in_specs=[
          pl.BlockSpec((VB, H), lambda i: (i, 0)),
          pl.BlockSpec((VB, GD), lambda i: (i, 0)),
          pl.BlockSpec((H, H), lambda i: (0, 0)),
          pl.BlockSpec((GD, H), lambda i: (0, 0)),
          pl.BlockSpec((1, H), lambda i: (0, 0)),
      ],
      out_specs=pl.BlockSpec((VB, H), lambda i: (i, 0)),
      out_shape=jax.ShapeDtypeStruct((V, H), jnp.float32),
  )(word_emb, glyph_emb, wa_bf, gw_bf, constv)


def _main_body(pg_ref, pid_ref, wb_ref, t01_ref, convb_ref, pos_ref,
               lng_ref, lnb_ref, out_ref):
  pid = pid_ref[0]
  iot = lax.broadcasted_iota(jnp.int32, (S, 2 * PV), 1)
  t01 = t01_ref[...]
  m = None
  for j in range(PL - 1):
    th = ((pid[:, j:j + 1] == iot)
          | (pid[:, j + 1:j + 2] + PV == iot)).astype(jnp.bfloat16)
    x = jnp.dot(th, t01, preferred_element_type=jnp.float32)
    m = x if m is None else jnp.maximum(m, x)
  py = m + convb_ref[...]
  acc = pg_ref[...] + jnp.dot(py.astype(jnp.bfloat16), wb_ref[...],
                              preferred_element_type=jnp.float32)
  acc = acc + pos_ref[...]
  mu = jnp.mean(acc, axis=1, keepdims=True)
  d = acc - mu
  var = jnp.mean(d * d, axis=1, keepdims=True)
  out_ref[...] = d * lax.rsqrt(var + EPS) * lng_ref[...] + lnb_ref[...]


def _main(pg, pid2d, wb_bf, t01_bf, convb, pos_emb, lng, lnb):
  return pl.pallas_call(
      _main_body,
      grid=(N // S,),
      in_specs=[
          pl.BlockSpec((S, H), lambda i: (i, 0)),
          pl.BlockSpec((1, S, PL), lambda i: (i, 0, 0)),
          pl.BlockSpec((H, H), lambda i: (0, 0)),
          pl.BlockSpec((2 * PV, H), lambda i: (0, 0)),
          pl.BlockSpec((1, H), lambda i: (0, 0)),
          pl.BlockSpec((S, H), lambda i: (0, 0)),
          pl.BlockSpec((1, H), lambda i: (0, 0)),
          pl.BlockSpec((1, H), lambda i: (0, 0)),
      ],
      out_specs=pl.BlockSpec((S, H), lambda i: (i, 0)),
      out_shape=jax.ShapeDtypeStruct((N, H), jnp.float32),
  )(pg, pid2d, wb_bf, t01_bf, convb, pos_emb, lng, lnb)


def kernel(input_ids, pinyin_ids, word_emb, pinyin_emb, conv_w, conv_b,
           glyph_emb, glyph_map_w, glyph_map_b, map_fc_w, map_fc_b,
           pos_emb, tt_emb, ln_g, ln_b):
  gw_bf, t01_bf, constv = _prep(
      glyph_map_w, map_fc_w[2 * H:], glyph_map_b.reshape(1, H),
      map_fc_b.reshape(1, H), tt_emb[0:1], pinyin_emb, conv_w[0], conv_w[1])

  wa_bf = map_fc_w[:H].astype(jnp.bfloat16)
  wb_bf = map_fc_w[H:2 * H].astype(jnp.bfloat16)

  ptab = _proj(word_emb, glyph_emb, wa_bf, gw_bf, constv)
  pg = _sc_gather(ptab, input_ids.reshape(N // CHUNK, CHUNK))

  out = _main(pg, pinyin_ids, wb_bf, t01_bf,
              conv_b.reshape(1, H), pos_emb,
              ln_g.reshape(1, H), ln_b.reshape(1, H))
  return out.reshape(B, S, H)


# speedup vs baseline: 5.5379x; 1.6943x over previous
import functools

import jax
import jax.numpy as jnp
from jax import lax
from jax.experimental import pallas as pl
from jax.experimental.pallas import tpu as pltpu
from jax.experimental.pallas import tpu_sc as plsc

B, S = 32, 512
N = B * S
H = 768
HP = H // 2
V = 23236
GD = 1728
PV, PE, PL = 32, 128, 8
EPS = 1e-12

NC, NS = 2, 16
NW = NC * NS
TOK_PER_W = N // NW
CHUNK = 128
NCH = TOK_PER_W // CHUNK

VB = 1024
NVB = (V + VB - 1) // VB


def _sc_gather(table, ids):
  mesh = plsc.VectorSubcoreMesh(core_axis_name="c", subcore_axis_name="s")

  @functools.partial(
      pl.kernel, mesh=mesh,
      out_type=jax.ShapeDtypeStruct((N, HP), jnp.float32),
      scratch_types=[
          pltpu.VMEM((NCH, CHUNK), jnp.int32),
          pltpu.VMEM((CHUNK, HP), jnp.float32),
          pltpu.VMEM((CHUNK, HP), jnp.float32),
          pltpu.SemaphoreType.DMA,
          pltpu.SemaphoreType.DMA,
      ],
  )
  def k(table_hbm, ids_hbm, out_hbm, idx_v, rows0, rows1, gsem, wsem):
    wid = lax.axis_index("s") * NC + lax.axis_index("c")
    base = wid * TOK_PER_W
    pltpu.sync_copy(ids_hbm.at[pl.ds(wid * NCH, NCH)], idx_v)
    bufs = (rows0, rows1)
    g = [None] * NCH
    w = [None] * NCH
    g[0] = pltpu.async_copy(table_hbm.at[idx_v.at[0]], rows0, gsem)
    for c in range(NCH):
      buf = bufs[c % 2]
      g[c].wait()
      if c + 1 < NCH:
        if c >= 1:
          w[c - 1].wait()
        g[c + 1] = pltpu.async_copy(
            table_hbm.at[idx_v.at[c + 1]], bufs[(c + 1) % 2], gsem)
      w[c] = pltpu.async_copy(
          buf, out_hbm.at[pl.ds(base + c * CHUNK, CHUNK)], wsem)
    if NCH >= 2:
      w[NCH - 2].wait()
    w[NCH - 1].wait()

  return k(table, ids)


def _prep_body(gmw_ref, wc_ref, gmb_ref, fcb_ref, tt0_ref, pemb_ref,
               cw0_ref, cw1_ref, gw_ref, t01_ref, const_ref):
  gw = jnp.dot(gmw_ref[...].astype(jnp.bfloat16), wc_ref[...].astype(jnp.bfloat16),
               preferred_element_type=jnp.float32)
  gw_ref[...] = gw.astype(jnp.bfloat16)
  t0 = jnp.dot(pemb_ref[...], cw0_ref[...], preferred_element_type=jnp.float32)
  t1 = jnp.dot(pemb_ref[...], cw1_ref[...], preferred_element_type=jnp.float32)
  t01_ref[...] = jnp.concatenate([t0, t1], axis=0).astype(jnp.bfloat16)
  const_ref[...] = (jnp.dot(gmb_ref[...], wc_ref[...],
                            preferred_element_type=jnp.float32)
                    + fcb_ref[...] + tt0_ref[...])


def _prep(glyph_map_w, wc, gmb, fcb, tt0, pinyin_emb, cw0, cw1):
  return pl.pallas_call(
      _prep_body,
      out_shape=[jax.ShapeDtypeStruct((GD, H), jnp.bfloat16),
                 jax.ShapeDtypeStruct((2 * PV, H), jnp.bfloat16),
                 jax.ShapeDtypeStruct((1, H), jnp.float32)],
  )(glyph_map_w, wc, gmb, fcb, tt0, pinyin_emb, cw0, cw1)


def _proj_body(we_ref, get_ref, wa_ref, gw_ref, const_ref, out_ref):
  acc = jnp.dot(we_ref[...].astype(jnp.bfloat16), wa_ref[...],
                preferred_element_type=jnp.float32)
  acc = acc + lax.dot_general(get_ref[...].astype(jnp.bfloat16), gw_ref[...],
                              (((0,), (0,)), ((), ())),
                              preferred_element_type=jnp.float32)
  acc = acc + const_ref[...]
  lo = lax.bitcast_convert_type(acc[:, :HP], jnp.uint32)
  hi = lax.bitcast_convert_type(acc[:, HP:], jnp.uint32)
  lo16 = ((lo + jnp.uint32(0x8000)) >> 16) & jnp.uint32(0xFFFF)
  hi16 = (hi + jnp.uint32(0x8000)) & jnp.uint32(0xFFFF0000)
  out_ref[...] = lax.bitcast_convert_type(lo16 | hi16, jnp.float32)


def _proj(word_emb, glyph_emb_t, wa_bf, gw_bf, constv):
  return pl.pallas_call(
      _proj_body,
      grid=(NVB,),
      in_specs=[
          pl.BlockSpec((VB, H), lambda i: (i, 0)),
          pl.BlockSpec((GD, VB), lambda i: (0, i)),
          pl.BlockSpec((H, H), lambda i: (0, 0)),
          pl.BlockSpec((GD, H), lambda i: (0, 0)),
          pl.BlockSpec((1, H), lambda i: (0, 0)),
      ],
      out_specs=pl.BlockSpec((VB, HP), lambda i: (i, 0)),
      out_shape=jax.ShapeDtypeStruct((V, HP), jnp.float32),
  )(word_emb, glyph_emb_t, wa_bf, gw_bf, constv)


def _main_body(pg_ref, pid_ref, wb_ref, t01_ref, convb_ref, pos_ref,
               lng_ref, lnb_ref, out_ref):
  pid = pid_ref[0]
  iot = lax.broadcasted_iota(jnp.int32, (S, 2 * PV), 1)
  t01 = t01_ref[...]
  m = None
  for j in range(PL - 1):
    th = ((pid[:, j:j + 1] == iot)
          | (pid[:, j + 1:j + 2] + PV == iot)).astype(jnp.bfloat16)
    x = jnp.dot(th, t01, preferred_element_type=jnp.float32)
    m = x if m is None else jnp.maximum(m, x)
  py = m + convb_ref[...]
  gbits = lax.bitcast_convert_type(pg_ref[...], jnp.uint32)
  plo = lax.bitcast_convert_type(gbits << 16, jnp.float32)
  phi = lax.bitcast_convert_type(gbits & jnp.uint32(0xFFFF0000), jnp.float32)
  acc = jnp.concatenate([plo, phi], axis=1)
  acc = acc + jnp.dot(py.astype(jnp.bfloat16), wb_ref[...],
                      preferred_element_type=jnp.float32)
  acc = acc + pos_ref[...]
  mu = jnp.mean(acc, axis=1, keepdims=True)
  d = acc - mu
  var = jnp.mean(d * d, axis=1, keepdims=True)
  out_ref[...] = d * lax.rsqrt(var + EPS) * lng_ref[...] + lnb_ref[...]


def _main(pg, pid3d, wb_bf, t01_bf, convb, pos_emb, lng, lnb):
  return pl.pallas_call(
      _main_body,
      grid=(N // S,),
      in_specs=[
          pl.BlockSpec((S, HP), lambda i: (i, 0)),
          pl.BlockSpec((1, S, PL), lambda i: (i, 0, 0)),
          pl.BlockSpec((H, H), lambda i: (0, 0)),
          pl.BlockSpec((2 * PV, H), lambda i: (0, 0)),
          pl.BlockSpec((1, H), lambda i: (0, 0)),
          pl.BlockSpec((S, H), lambda i: (0, 0)),
          pl.BlockSpec((1, H), lambda i: (0, 0)),
          pl.BlockSpec((1, H), lambda i: (0, 0)),
      ],
      out_specs=pl.BlockSpec((S, H), lambda i: (i, 0)),
      out_shape=jax.ShapeDtypeStruct((N, H), jnp.float32),
  )(pg, pid3d, wb_bf, t01_bf, convb, pos_emb, lng, lnb)


def kernel(input_ids, pinyin_ids, word_emb, pinyin_emb, conv_w, conv_b,
           glyph_emb, glyph_map_w, glyph_map_b, map_fc_w, map_fc_b,
           pos_emb, tt_emb, ln_g, ln_b):
  gw_bf, t01_bf, constv = _prep(
      glyph_map_w, map_fc_w[2 * H:], glyph_map_b.reshape(1, H),
      map_fc_b.reshape(1, H), tt_emb[0:1], pinyin_emb, conv_w[0], conv_w[1])

  wa_bf = map_fc_w[:H].astype(jnp.bfloat16)
  wb_bf = map_fc_w[H:2 * H].astype(jnp.bfloat16)

  ptab = _proj(word_emb, glyph_emb.T, wa_bf, gw_bf, constv)
  pg = _sc_gather(ptab, input_ids.reshape(N // CHUNK, CHUNK))

  out = _main(pg, pinyin_ids, wb_bf, t01_bf,
              conv_b.reshape(1, H), pos_emb,
              ln_g.reshape(1, H), ln_b.reshape(1, H))
  return out.reshape(B, S, H)
